# Initial kernel scaffold; baseline (speedup 1.0000x reference)
import jax
import jax.numpy as jnp
from jax.experimental import pallas as pl


def kernel(x, params):
    raise NotImplementedError("write your pallas kernel here")


# speedup vs baseline: 2.8537x; 2.8537x over previous
import functools

import jax
import jax.numpy as jnp
from jax.experimental import pallas as pl

NUM_K = 16
DECIM = 4
D_IN = 6
NUM_CLASSES = 7


def _knn_body(keys_ref, q_ref, idx_ref, d2_ref, *, k, n):
    kT = keys_ref[0]
    qT = q_ref[0]
    sq_k = jnp.sum(kT * kT, axis=0, keepdims=True)
    sq_q = jnp.sum(qT * qT, axis=0)[:, None]
    cross = jax.lax.dot_general(
        qT, kT, dimension_numbers=(((0,), (0,)), ((), ())),
        preferred_element_type=jnp.float32)
    d2 = sq_q + sq_k - 2.0 * cross
    t = d2.shape[0]
    iota = jax.lax.broadcasted_iota(jnp.int32, (t, n), 1)
    big = jnp.float32(jnp.inf)
    for j in range(k):
        m = jnp.min(d2, axis=1, keepdims=True)
        sel = jnp.where(d2 == m, iota, n)
        ij = jnp.min(sel, axis=1)
        idx_ref[0, :, j] = ij
        d2_ref[0, :, j] = m[:, 0]
        d2 = jnp.where(iota == ij[:, None], big, d2)


def _knn_pallas(pos, k):
    b, n, _ = pos.shape
    post = jnp.transpose(pos, (0, 2, 1))
    t = min(n, 256)
    nt = n // t
    grid = (b, nt)
    kern = functools.partial(_knn_body, k=k, n=n)
    idx, d2 = pl.pallas_call(
        kern,
        grid=grid,
        in_specs=[
            pl.BlockSpec((1, 3, n), lambda bb, tt: (bb, 0, 0)),
            pl.BlockSpec((1, 3, t), lambda bb, tt: (bb, 0, tt)),
        ],
        out_specs=[
            pl.BlockSpec((1, t, k), lambda bb, tt: (bb, tt, 0)),
            pl.BlockSpec((1, t, k), lambda bb, tt: (bb, tt, 0)),
        ],
        out_shape=[
            jax.ShapeDtypeStruct((b, n, k), jnp.int32),
            jax.ShapeDtypeStruct((b, n, k), jnp.float32),
        ],
    )(post, post)
    return idx, d2


def _nn1_body(keys_ref, q_ref, idx_ref, *, ns):
    kT = keys_ref[0]
    qT = q_ref[0]
    sq_k = jnp.sum(kT * kT, axis=0, keepdims=True)
    sq_q = jnp.sum(qT * qT, axis=0)[:, None]
    cross = jax.lax.dot_general(
        qT, kT, dimension_numbers=(((0,), (0,)), ((), ())),
        preferred_element_type=jnp.float32)
    d2 = sq_q + sq_k - 2.0 * cross
    t = d2.shape[0]
    m = jnp.min(d2, axis=1, keepdims=True)
    iota = jax.lax.broadcasted_iota(jnp.int32, (t, ns), 1)
    ij = jnp.min(jnp.where(d2 == m, iota, ns), axis=1)
    idx_ref[0, 0] = ij


def _nn1_pallas(pos_q, pos_s):
    b, nq, _ = pos_q.shape
    ns = pos_s.shape[1]
    post_q = jnp.transpose(pos_q, (0, 2, 1))
    post_s = jnp.transpose(pos_s, (0, 2, 1))
    t = min(nq, 512)
    nt = nq // t
    kern = functools.partial(_nn1_body, ns=ns)
    idx = pl.pallas_call(
        kern,
        grid=(b, nt),
        in_specs=[
            pl.BlockSpec((1, 3, ns), lambda bb, tt: (bb, 0, 0)),
            pl.BlockSpec((1, 3, t), lambda bb, tt: (bb, 0, tt)),
        ],
        out_specs=pl.BlockSpec((1, 1, t), lambda bb, tt: (bb * nt + tt, 0, 0)),
        out_shape=jax.ShapeDtypeStruct((b * nt, 1, t), jnp.int32),
    )(post_s, post_q)
    return idx.reshape(b, nq)


def _batchnorm(x, gamma, beta, eps=1e-6):
    mean = jnp.mean(x, axis=(0, 2, 3), keepdims=True)
    var = jnp.var(x, axis=(0, 2, 3), keepdims=True)
    return gamma[None, :, None, None] * (x - mean) / jnp.sqrt(var + eps) + beta[None, :, None, None]


def _smlp(p, x, act=None):
    y = jnp.einsum('bcnk,oc->bonk', x, p['W']) + p['b'][None, :, None, None]
    if 'gamma' in p:
        y = _batchnorm(y, p['gamma'], p['beta'])
    if act is not None:
        y = act(y)
    return y


def _gather_nbrs(pos, idx):
    g = jax.vmap(lambda p, i: p[i])(pos, idx)
    return jnp.transpose(g, (0, 3, 1, 2))


def _lse(p, pos, features, idx, dist2):
    B, N, Kn = idx.shape
    ext = jnp.broadcast_to(jnp.transpose(pos, (0, 2, 1))[:, :, :, None], (B, 3, N, Kn))
    nbrs = _gather_nbrs(pos, idx)
    concat = jnp.concatenate([ext, nbrs, ext - nbrs, dist2[:, None, :, :]], axis=1)
    enc = _smlp(p, concat, jax.nn.relu)
    feat = jnp.broadcast_to(features, (B, features.shape[1], N, Kn))
    return jnp.concatenate([enc, feat], axis=1)


def _att_pool(score_W, mlp_p, x):
    xp = jnp.transpose(x, (0, 2, 3, 1))
    s = jax.nn.softmax(jnp.einsum('bnkc,oc->bnko', xp, score_W), axis=-2)
    scores = jnp.transpose(s, (0, 3, 1, 2))
    feat = jnp.sum(scores * x, axis=-1, keepdims=True)
    return _smlp(mlp_p, feat, jax.nn.relu)


def _lfa(p, pos, features):
    idx, _ = _knn_pallas(pos, NUM_K)
    nbrs = _gather_nbrs(pos, idx)
    B, N, _ = pos.shape
    ext = jnp.broadcast_to(jnp.transpose(pos, (0, 2, 1))[:, :, :, None], (B, 3, N, NUM_K))
    dist2 = jnp.sum((nbrs - ext) ** 2, axis=1)
    x = _smlp(p['mlp1'], features, lambda v: jax.nn.leaky_relu(v, 0.2))
    x = _lse(p['lse1'], pos, x, idx, dist2)
    x = _att_pool(p['pool1_score'], p['pool1_mlp'], x)
    x = _lse(p['lse2'], pos, x, idx, dist2)
    x = _att_pool(p['pool2_score'], p['pool2_mlp'], x)
    return jax.nn.leaky_relu(_smlp(p['mlp2'], x) + _smlp(p['shortcut'], features), 0.01)


def kernel(x, params):
    B, N, _ = x.shape
    coords = x[..., :3]
    h = jnp.einsum('bnd,od->bno', x, params['fc_start']['W']) + params['fc_start']['b']
    h = jnp.transpose(h, (0, 2, 1))[:, :, :, None]
    h = _batchnorm(h, params['bn_start']['gamma'], params['bn_start']['beta'])
    h = jax.nn.leaky_relu(h, 0.2)
    ratio = 1
    stack = []
    for p in params['enc']:
        n_cur = N // ratio
        h = _lfa(p, coords[:, :n_cur], h)
        stack.append(h)
        ratio *= DECIM
        h = h[:, :, : N // ratio]
    h = _smlp(params['mid'], h, jax.nn.relu)
    for p in params['dec']:
        n_coarse = N // ratio
        ratio //= DECIM
        n_fine = N // ratio
        idx1 = _nn1_pallas(coords[:, :n_fine], coords[:, :n_coarse])
        up = jax.vmap(lambda hb, ib: hb[:, ib, :])(h, idx1)
        skip = stack.pop()
        h = _smlp(p, jnp.concatenate([up, skip], axis=1), jax.nn.relu)
    h = _smlp(params['fc_end1'], h, jax.nn.relu)
    h = _smlp(params['fc_end2'], h, jax.nn.relu)
    h = _smlp(params['fc_out'], h)
    return jnp.squeeze(h, -1)


# speedup vs baseline: 3.5132x; 1.2311x over previous
import functools

import jax
import jax.numpy as jnp
from jax.experimental import pallas as pl
from jax.experimental.pallas import tpu as pltpu

NUM_K = 16
DECIM = 4
D_IN = 6
NUM_CLASSES = 7
_T = 256
_HI = jax.lax.Precision.HIGHEST


def _act(v, slope):
    if slope is None:
        return v
    return jnp.where(v >= 0, v, slope * v)


def _mm(a, b, prec=None):
    return jax.lax.dot_general(a, b, (((1,), (0,)), ((), ())), precision=prec)


def _rowsums(y):
    return (jnp.sum(y, axis=1, keepdims=True),
            jnp.sum(y * y, axis=1, keepdims=True))


def _bn_vec(p, s, q, cnt, eps=1e-6):
    m = s[:, 0] / cnt
    var = q[:, 0] / cnt - m * m
    inv = p['gamma'] / jnp.sqrt(var + eps)
    return inv[:, None], (p['beta'] - inv * m)[:, None]


def _const_spec(x):
    return pl.BlockSpec(x.shape, lambda *a: (0,) * x.ndim)


def _knng_body(keys_ref, q_ref, nbrs_ref, d2_ref, *, n, tq, nb, k):
    for b in range(nb):
        kT = keys_ref[:, b * n:(b + 1) * n]
        qT = q_ref[:, b * tq:(b + 1) * tq]
        sq_k = jnp.sum(kT * kT, axis=0, keepdims=True)
        sq_q = jnp.sum(qT * qT, axis=0)[:, None]
        cross = jax.lax.dot_general(
            qT, kT, (((0,), (0,)), ((), ())),
            preferred_element_type=jnp.float32)
        d2 = sq_q + sq_k - 2.0 * cross
        iota = jax.lax.broadcasted_iota(jnp.int32, (tq, n), 1)
        big = jnp.float32(jnp.inf)
        for j in range(k):
            m = jnp.min(d2, axis=1, keepdims=True)
            ij = jnp.min(jnp.where(d2 == m, iota, n), axis=1)
            sel = iota == ij[:, None]
            nbr = jax.lax.dot_general(
                kT, sel.astype(jnp.float32), (((1,), (1,)), ((), ())),
                precision=_HI)
            nbrs_ref[:, j, b * tq:(b + 1) * tq] = nbr
            diff = qT - nbr
            d2_ref[j, b * tq:(b + 1) * tq] = jnp.sum(diff * diff, axis=0)
            d2 = jnp.where(sel, big, d2)


def _knng_call(pos_flat, b, n):
    s = b * n
    out_shape = [
        jax.ShapeDtypeStruct((3, NUM_K, s), jnp.float32),
        jax.ShapeDtypeStruct((NUM_K, s), jnp.float32),
    ]
    if n >= _T:
        nt = n // _T
        kern = functools.partial(_knng_body, n=n, tq=_T, nb=1, k=NUM_K)
        return pl.pallas_call(
            kern,
            grid=(b, nt),
            in_specs=[
                pl.BlockSpec((3, n), lambda bb, tt: (0, bb)),
                pl.BlockSpec((3, _T), lambda bb, tt: (0, bb * nt + tt)),
            ],
            out_specs=[
                pl.BlockSpec((3, NUM_K, _T), lambda bb, tt: (0, 0, bb * nt + tt)),
                pl.BlockSpec((NUM_K, _T), lambda bb, tt: (0, bb * nt + tt)),
            ],
            out_shape=out_shape,
        )(pos_flat, pos_flat)
    kern = functools.partial(_knng_body, n=n, tq=n, nb=b, k=NUM_K)
    return pl.pallas_call(
        kern,
        grid=(1,),
        in_specs=[_const_spec(pos_flat), _const_spec(pos_flat)],
        out_specs=[
            pl.BlockSpec((3, NUM_K, s), lambda t: (0, 0, 0)),
            pl.BlockSpec((NUM_K, s), lambda t: (0, 0)),
        ],
        out_shape=out_shape,
    )(pos_flat, pos_flat)


def _gaff_body(*refs, pre, post, two, stats):
    i = 0
    x_ref = refs[i]; i += 1
    if pre is not None:
        sc_ref, sh_ref = refs[i], refs[i + 1]; i += 2
        z = _act(sc_ref[...] * x_ref[...] + sh_ref[...], pre)
    else:
        z = x_ref[...]
    a_ref = refs[i]; i += 1
    y = _mm(a_ref[...], z)
    if two:
        x2_ref, a2_ref = refs[i], refs[i + 1]; i += 2
        y = y + _mm(a2_ref[...], x2_ref[...])
    c_ref = refs[i]; i += 1
    y = _act(y + c_ref[...], post)
    y_ref = refs[i]; i += 1
    y_ref[...] = y
    if stats:
        s_ref, q_ref = refs[i], refs[i + 1]

        @pl.when(pl.program_id(0) == 0)
        def _():
            s_ref[...] = jnp.zeros_like(s_ref)
            q_ref[...] = jnp.zeros_like(q_ref)
        s, q = _rowsums(y)
        s_ref[...] += s
        q_ref[...] += q


def _gaff_call(x, a, c, pre=None, prevec=None, post=None, x2=None, a2=None,
               stats=False):
    cch, s = x.shape
    o = a.shape[0]
    t = min(_T, s)
    two = x2 is not None
    kern = functools.partial(_gaff_body, pre=pre, post=post, two=two,
                             stats=stats)
    in_specs = [pl.BlockSpec((cch, t), lambda tt: (0, tt))]
    args = [x]
    if pre is not None:
        in_specs += [_const_spec(prevec[0]), _const_spec(prevec[1])]
        args += [prevec[0], prevec[1]]
    in_specs.append(_const_spec(a))
    args.append(a)
    if two:
        in_specs += [pl.BlockSpec((x2.shape[0], t), lambda tt: (0, tt)),
                     _const_spec(a2)]
        args += [x2, a2]
    in_specs.append(_const_spec(c))
    args.append(c)
    out_specs = [pl.BlockSpec((o, t), lambda tt: (0, tt))]
    out_shape = [jax.ShapeDtypeStruct((o, s), jnp.float32)]
    if stats:
        out_specs += [pl.BlockSpec((o, 1), lambda tt: (0, 0)),
                      pl.BlockSpec((o, 1), lambda tt: (0, 0))]
        out_shape += [jax.ShapeDtypeStruct((o, 1), jnp.float32),
                      jax.ShapeDtypeStruct((o, 1), jnp.float32)]
    res = pl.pallas_call(
        kern,
        grid=(s // t,),
        in_specs=in_specs,
        out_specs=out_specs,
        out_shape=out_shape,
    )(*args)
    return res if stats else res[0]


def _norm_body(x_ref, sc_ref, sh_ref, y_ref, *, slope):
    y_ref[...] = _act(sc_ref[...] * x_ref[...] + sh_ref[...], slope)


def _norm_call(x, sc, sh, slope):
    cch, s = x.shape
    t = min(_T, s)
    kern = functools.partial(_norm_body, slope=slope)
    return pl.pallas_call(
        kern,
        grid=(s // t,),
        in_specs=[pl.BlockSpec((cch, t), lambda tt: (0, tt)),
                  _const_spec(sc), _const_spec(sh)],
        out_specs=pl.BlockSpec((cch, t), lambda tt: (0, tt)),
        out_shape=jax.ShapeDtypeStruct((cch, s), jnp.float32),
    )(x, sc, sh)


def _prestats_body(pos_ref, nbrs_ref, d2_ref, feat_ref,
                   wl1_ref, bl1_ref, wl2_ref, bl2_ref, wsc_ref, bsc_ref,
                   s1_ref, q1_ref, s2_ref, q2_ref, ssc_ref, qsc_ref):
    @pl.when(pl.program_id(0) == 0)
    def _():
        for r in (s1_ref, q1_ref, s2_ref, q2_ref, ssc_ref, qsc_ref):
            r[...] = jnp.zeros_like(r)
    ext = pos_ref[...]
    wl1, bl1 = wl1_ref[...], bl1_ref[...]
    wl2, bl2 = wl2_ref[...], bl2_ref[...]
    s1 = jnp.zeros_like(s1_ref)
    q1 = jnp.zeros_like(q1_ref)
    s2 = jnp.zeros_like(s2_ref)
    q2 = jnp.zeros_like(q2_ref)
    for kk in range(NUM_K):
        nb = nbrs_ref[:, kk, :]
        gk = jnp.concatenate([ext, nb, ext - nb, d2_ref[kk:kk + 1, :]], axis=0)
        e1 = _mm(wl1, gk) + bl1
        a, b = _rowsums(e1)
        s1 += a
        q1 += b
        e2 = _mm(wl2, gk) + bl2
        a, b = _rowsums(e2)
        s2 += a
        q2 += b
    s1_ref[...] += s1
    q1_ref[...] += q1
    s2_ref[...] += s2
    q2_ref[...] += q2
    ysc = _mm(wsc_ref[...], feat_ref[...]) + bsc_ref[...]
    a, b = _rowsums(ysc)
    ssc_ref[...] += a
    qsc_ref[...] += b


def _prestats_call(pos_flat, nbrs, d2, feats, wl1, bl1, wl2, bl2, wsc, bsc):
    fch, s = feats.shape
    eg = wl1.shape[0]
    d2ch = wsc.shape[0]
    vec = lambda ch: jax.ShapeDtypeStruct((ch, 1), jnp.float32)
    cvec = lambda ch: pl.BlockSpec((ch, 1), lambda tt: (0, 0))
    return pl.pallas_call(
        _prestats_body,
        grid=(s // _T,),
        in_specs=[
            pl.BlockSpec((3, _T), lambda tt: (0, tt)),
            pl.BlockSpec((3, NUM_K, _T), lambda tt: (0, 0, tt)),
            pl.BlockSpec((NUM_K, _T), lambda tt: (0, tt)),
            pl.BlockSpec((fch, _T), lambda tt: (0, tt)),
            _const_spec(wl1), _const_spec(bl1),
            _const_spec(wl2), _const_spec(bl2),
            _const_spec(wsc), _const_spec(bsc),
        ],
        out_specs=[cvec(eg), cvec(eg), cvec(eg), cvec(eg),
                   cvec(d2ch), cvec(d2ch)],
        out_shape=[vec(eg), vec(eg), vec(eg), vec(eg), vec(d2ch), vec(d2ch)],
    )(pos_flat, nbrs, d2, feats, wl1, bl1, wl2, bl2, wsc, bsc)


def _pool_body(pos_ref, nbrs_ref, d2_ref, feat_ref, f1_ref, f2_ref,
               wl_ref, bl_ref, scl_ref, shl_ref, ws_ref, wp_ref, bp_ref,
               y_ref, s_ref, q_ref, xc_ref, *, mode, slope_f):
    ext = pos_ref[...]
    if mode == 'mm':
        xf = _act(_mm(f1_ref[...], feat_ref[...]) + f2_ref[...], slope_f)
    else:
        xf = _act(f1_ref[...] * feat_ref[...] + f2_ref[...], 0.0)
    wl, bl = wl_ref[...], bl_ref[...]
    scl, shl = scl_ref[...], shl_ref[...]
    ws = ws_ref[...]
    d = ws.shape[0]
    t = ext.shape[1]
    m = jnp.full((d, t), -jnp.inf, jnp.float32)
    for kk in range(NUM_K):
        nb = nbrs_ref[:, kk, :]
        gk = jnp.concatenate([ext, nb, ext - nb, d2_ref[kk:kk + 1, :]], axis=0)
        enc = _act(scl * (_mm(wl, gk) + bl) + shl, 0.0)
        xk = jnp.concatenate([enc, xf], axis=0)
        xc_ref[kk] = xk
        m = jnp.maximum(m, _mm(ws, xk))
    ssum = jnp.zeros((d, t), jnp.float32)
    acc = jnp.zeros((d, t), jnp.float32)
    for kk in range(NUM_K):
        xk = xc_ref[kk]
        e = jnp.exp(_mm(ws, xk) - m)
        ssum += e
        acc += e * xk
    pooled = acc / ssum
    y = _mm(wp_ref[...], pooled) + bp_ref[...]
    y_ref[...] = y

    @pl.when(pl.program_id(0) == 0)
    def _():
        s_ref[...] = jnp.zeros_like(s_ref)
        q_ref[...] = jnp.zeros_like(q_ref)
    s, q = _rowsums(y)
    s_ref[...] += s
    q_ref[...] += q


def _pool_call(pos_flat, nbrs, d2, feat, f1, f2, wl, bl, scl, shl, ws,
               wp, bp, mode, slope_f=None):
    fch, s = feat.shape
    d = ws.shape[0]
    op = wp.shape[0]
    kern = functools.partial(_pool_body, mode=mode, slope_f=slope_f)
    return pl.pallas_call(
        kern,
        grid=(s // _T,),
        in_specs=[
            pl.BlockSpec((3, _T), lambda tt: (0, tt)),
            pl.BlockSpec((3, NUM_K, _T), lambda tt: (0, 0, tt)),
            pl.BlockSpec((NUM_K, _T), lambda tt: (0, tt)),
            pl.BlockSpec((fch, _T), lambda tt: (0, tt)),
            _const_spec(f1), _const_spec(f2),
            _const_spec(wl), _const_spec(bl),
            _const_spec(scl), _const_spec(shl),
            _const_spec(ws), _const_spec(wp), _const_spec(bp),
        ],
        out_specs=[
            pl.BlockSpec((op, _T), lambda tt: (0, tt)),
            pl.BlockSpec((op, 1), lambda tt: (0, 0)),
            pl.BlockSpec((op, 1), lambda tt: (0, 0)),
        ],
        out_shape=[
            jax.ShapeDtypeStruct((op, s), jnp.float32),
            jax.ShapeDtypeStruct((op, 1), jnp.float32),
            jax.ShapeDtypeStruct((op, 1), jnp.float32),
        ],
        scratch_shapes=[pltpu.VMEM((NUM_K, d, _T), jnp.float32)],
    )(pos_flat, nbrs, d2, feat, f1, f2, wl, bl, scl, shl, ws, wp, bp)


def _dual_body(y2_ref, scp_ref, shp_ref, wm_ref, bm_ref, fin_ref,
               wsc_ref, bsc_ref, scs_ref, shs_ref, h_ref):
    p2 = _act(scp_ref[...] * y2_ref[...] + shp_ref[...], 0.0)
    y = _mm(wm_ref[...], p2) + bm_ref[...]
    ysc = scs_ref[...] * (_mm(wsc_ref[...], fin_ref[...]) + bsc_ref[...]) \
        + shs_ref[...]
    h_ref[...] = _act(y + ysc, 0.01)


def _dual_call(y2, scp, shp, wm, bm, fin, wsc, bsc, scs, shs):
    dch, s = y2.shape
    fch = fin.shape[0]
    o = wm.shape[0]
    return pl.pallas_call(
        _dual_body,
        grid=(s // _T,),
        in_specs=[
            pl.BlockSpec((dch, _T), lambda tt: (0, tt)),
            _const_spec(scp), _const_spec(shp),
            _const_spec(wm), _const_spec(bm),
            pl.BlockSpec((fch, _T), lambda tt: (0, tt)),
            _const_spec(wsc), _const_spec(bsc),
            _const_spec(scs), _const_spec(shs),
        ],
        out_specs=pl.BlockSpec((o, _T), lambda tt: (0, tt)),
        out_shape=jax.ShapeDtypeStruct((o, s), jnp.float32),
    )(y2, scp, shp, wm, bm, fin, wsc, bsc, scs, shs)


def _up_body(keys_ref, q_ref, h_ref, *rest, nc, tq, nb, norm):
    if norm:
        sc_ref, sh_ref, up_ref = rest
        h_all = _act(sc_ref[...] * h_ref[...] + sh_ref[...], 0.0)
    else:
        (up_ref,) = rest
        h_all = h_ref[...]
    for b in range(nb):
        kT = keys_ref[:, b * nc:(b + 1) * nc]
        qT = q_ref[:, b * tq:(b + 1) * tq]
        h_b = h_all[:, b * nc:(b + 1) * nc]
        sq_k = jnp.sum(kT * kT, axis=0, keepdims=True)
        sq_q = jnp.sum(qT * qT, axis=0)[:, None]
        cross = jax.lax.dot_general(
            qT, kT, (((0,), (0,)), ((), ())),
            preferred_element_type=jnp.float32)
        d2 = sq_q + sq_k - 2.0 * cross
        m = jnp.min(d2, axis=1, keepdims=True)
        iota = jax.lax.broadcasted_iota(jnp.int32, d2.shape, 1)
        ij = jnp.min(jnp.where(d2 == m, iota, nc), axis=1)
        sel = (iota == ij[:, None]).astype(jnp.float32)
        up = jax.lax.dot_general(
            h_b, sel, (((1,), (1,)), ((), ())), precision=_HI)
        up_ref[:, b * tq:(b + 1) * tq] = up


def _up_call(pos_s_flat, pos_q_flat, h, b, nc, nf, scsh=None):
    c = h.shape[0]
    sf = b * nf
    norm = scsh is not None
    extra = list(scsh) if norm else []
    out_shape = jax.ShapeDtypeStruct((c, sf), jnp.float32)
    if nf >= _T and nc % 128 == 0:
        nt = nf // _T
        kern = functools.partial(_up_body, nc=nc, tq=_T, nb=1, norm=norm)
        return pl.pallas_call(
            kern,
            grid=(b, nt),
            in_specs=[
                pl.BlockSpec((3, nc), lambda bb, tt: (0, bb)),
                pl.BlockSpec((3, _T), lambda bb, tt: (0, bb * nt + tt)),
                pl.BlockSpec((c, nc), lambda bb, tt: (0, bb)),
            ] + [_const_spec(e) for e in extra],
            out_specs=pl.BlockSpec((c, _T), lambda bb, tt: (0, bb * nt + tt)),
            out_shape=out_shape,
        )(pos_s_flat, pos_q_flat, h, *extra)
    kern = functools.partial(_up_body, nc=nc, tq=nf, nb=b, norm=norm)
    return pl.pallas_call(
        kern,
        grid=(1, 1),
        in_specs=[_const_spec(pos_s_flat), _const_spec(pos_q_flat),
                  _const_spec(h)] + [_const_spec(e) for e in extra],
        out_specs=pl.BlockSpec((c, sf), lambda bb, tt: (0, 0)),
        out_shape=out_shape,
    )(pos_s_flat, pos_q_flat, h, *extra)


def _b2(p):
    return p['b'][:, None]


def _lfa_fused(p, pos_flat, feats, b, n):
    s = b * n
    nbrs, d2 = _knng_call(pos_flat, b, n)
    s1, q1, s2, q2, ssc, qsc = _prestats_call(
        pos_flat, nbrs, d2, feats,
        p['lse1']['W'], _b2(p['lse1']), p['lse2']['W'], _b2(p['lse2']),
        p['shortcut']['W'], _b2(p['shortcut']))
    scl1, shl1 = _bn_vec(p['lse1'], s1, q1, s * NUM_K)
    scl2, shl2 = _bn_vec(p['lse2'], s2, q2, s * NUM_K)
    scsc, shsc = _bn_vec(p['shortcut'], ssc, qsc, s)
    y1, sp1, qp1 = _pool_call(
        pos_flat, nbrs, d2, feats, p['mlp1']['W'], _b2(p['mlp1']),
        p['lse1']['W'], _b2(p['lse1']), scl1, shl1,
        p['pool1_score'], p['pool1_mlp']['W'], _b2(p['pool1_mlp']),
        mode='mm', slope_f=0.2)
    scp1, shp1 = _bn_vec(p['pool1_mlp'], sp1, qp1, s)
    y2, sp2, qp2 = _pool_call(
        pos_flat, nbrs, d2, y1, scp1, shp1,
        p['lse2']['W'], _b2(p['lse2']), scl2, shl2,
        p['pool2_score'], p['pool2_mlp']['W'], _b2(p['pool2_mlp']),
        mode='norm')
    scp2, shp2 = _bn_vec(p['pool2_mlp'], sp2, qp2, s)
    return _dual_call(y2, scp2, shp2, p['mlp2']['W'], _b2(p['mlp2']),
                      feats, p['shortcut']['W'], _b2(p['shortcut']),
                      scsc, shsc)


def kernel(x, params):
    B, N, _ = x.shape
    S = B * N
    coords = jnp.transpose(x[..., :3], (0, 2, 1))
    xT = jnp.transpose(x, (2, 0, 1)).reshape(D_IN, S)

    h_raw, s0, q0 = _gaff_call(
        xT, params['fc_start']['W'], _b2(params['fc_start']), stats=True)
    sc0, sh0 = _bn_vec(params['bn_start'], s0, q0, S)
    h = _norm_call(h_raw, sc0, sh0, 0.2)

    ratio = 1
    stack = []
    pos_flats = []
    for p in params['enc']:
        n_cur = N // ratio
        pos_flat = jnp.transpose(
            coords[:, :, :n_cur], (1, 0, 2)).reshape(3, B * n_cur)
        pos_flats.append(pos_flat)
        h = _lfa_fused(p, pos_flat, h, B, n_cur)
        stack.append(h)
        ratio *= DECIM
        n_new = N // ratio
        ch = h.shape[0]
        h = h.reshape(ch, B, n_cur)[:, :, :n_new].reshape(ch, B * n_new)

    h = _gaff_call(h, params['mid']['W'], _b2(params['mid']), post=0.0)

    scsh = None
    prev_p = None
    for di, p in enumerate(params['dec']):
        n_coarse = N // ratio
        ratio //= DECIM
        n_fine = N // ratio
        pos_s = jnp.transpose(
            coords[:, :, :n_coarse], (1, 0, 2)).reshape(3, B * n_coarse)
        pos_q = pos_flats[3 - di]
        skip = stack.pop()
        up = _up_call(pos_s, pos_q, h, B, n_coarse, n_fine, scsh=scsh)
        c1 = up.shape[0]
        h, sd, qd = _gaff_call(
            up, p['W'][:, :c1], _b2(p), x2=skip, a2=p['W'][:, c1:],
            stats=True)
        scd, shd = _bn_vec(p, sd, qd, B * n_fine)
        scsh = (scd, shd)
        prev_p = p

    h, se, qe = _gaff_call(
        h, params['fc_end1']['W'], _b2(params['fc_end1']),
        pre=0.0, prevec=scsh, stats=True)
    sce, she = _bn_vec(params['fc_end1'], se, qe, S)
    h, se2, qe2 = _gaff_call(
        h, params['fc_end2']['W'], _b2(params['fc_end2']),
        pre=0.0, prevec=(sce, she), stats=True)
    sce2, she2 = _bn_vec(params['fc_end2'], se2, qe2, S)
    out = _gaff_call(
        h, params['fc_out']['W'], _b2(params['fc_out']),
        pre=0.0, prevec=(sce2, she2))
    return jnp.transpose(out.reshape(NUM_CLASSES, B, N), (1, 0, 2))


# speedup vs baseline: 6.2720x; 1.7853x over previous
import functools

import jax
import jax.numpy as jnp
from jax.experimental import pallas as pl
from jax.experimental.pallas import tpu as pltpu

NUM_K = 16
DECIM = 4
D_IN = 6
NUM_CLASSES = 7
_T = 256
_HI = jax.lax.Precision.HIGHEST


def _act(v, slope):
    if slope is None:
        return v
    return jnp.where(v >= 0, v, slope * v)


def _mm(a, b, prec=None):
    return jax.lax.dot_general(a, b, (((1,), (0,)), ((), ())), precision=prec)


def _rowsums(y):
    return (jnp.sum(y, axis=1, keepdims=True),
            jnp.sum(y * y, axis=1, keepdims=True))


def _bn_vec(p, s, q, cnt, eps=1e-6):
    m = s[:, 0] / cnt
    var = q[:, 0] / cnt - m * m
    inv = p['gamma'] / jnp.sqrt(var + eps)
    return inv[:, None], (p['beta'] - inv * m)[:, None]


def _const_spec(x):
    return pl.BlockSpec(x.shape, lambda *a: (0,) * x.ndim)


def _exact_gather(vals, sel):
    hi = vals.astype(jnp.bfloat16)
    r = vals - hi.astype(jnp.float32)
    mid = r.astype(jnp.bfloat16)
    lo = (r - mid.astype(jnp.float32)).astype(jnp.bfloat16)
    sel_bf = sel.astype(jnp.bfloat16)
    dn = (((1,), (1,)), ((), ()))
    g = jax.lax.dot_general(hi, sel_bf, dn,
                            preferred_element_type=jnp.float32)
    g = g + jax.lax.dot_general(mid, sel_bf, dn,
                                preferred_element_type=jnp.float32)
    g = g + jax.lax.dot_general(lo, sel_bf, dn,
                                preferred_element_type=jnp.float32)
    return g


def _knng_body(keys_ref, q_ref, nbrs_ref, d2_ref, *, n, tq, nb, k):
    for b in range(nb):
        kT = keys_ref[:, b * n:(b + 1) * n]
        qT = q_ref[:, b * tq:(b + 1) * tq]
        sq_k = jnp.sum(kT * kT, axis=0, keepdims=True)
        sq_q = jnp.sum(qT * qT, axis=0)[:, None]
        cross = jax.lax.dot_general(
            qT, kT, (((0,), (0,)), ((), ())),
            preferred_element_type=jnp.float32)
        d2 = sq_q + sq_k - 2.0 * cross
        iota = jax.lax.broadcasted_iota(jnp.int32, (tq, n), 1)
        big = jnp.float32(jnp.inf)
        for j in range(k):
            m = jnp.min(d2, axis=1, keepdims=True)
            ij = jnp.min(jnp.where(d2 == m, iota, n), axis=1)
            sel = iota == ij[:, None]
            nbr = _exact_gather(kT, sel)
            nbrs_ref[:, j, b * tq:(b + 1) * tq] = nbr
            diff = qT - nbr
            d2_ref[j, b * tq:(b + 1) * tq] = jnp.sum(diff * diff, axis=0)
            d2 = jnp.where(sel, big, d2)


def _knng_call(pos_flat, b, n):
    s = b * n
    out_shape = [
        jax.ShapeDtypeStruct((3, NUM_K, s), jnp.float32),
        jax.ShapeDtypeStruct((NUM_K, s), jnp.float32),
    ]
    if n >= _T:
        nt = n // _T
        kern = functools.partial(_knng_body, n=n, tq=_T, nb=1, k=NUM_K)
        return pl.pallas_call(
            kern,
            grid=(b, nt),
            in_specs=[
                pl.BlockSpec((3, n), lambda bb, tt: (0, bb)),
                pl.BlockSpec((3, _T), lambda bb, tt: (0, bb * nt + tt)),
            ],
            out_specs=[
                pl.BlockSpec((3, NUM_K, _T), lambda bb, tt: (0, 0, bb * nt + tt)),
                pl.BlockSpec((NUM_K, _T), lambda bb, tt: (0, bb * nt + tt)),
            ],
            out_shape=out_shape,
        )(pos_flat, pos_flat)
    kern = functools.partial(_knng_body, n=n, tq=n, nb=b, k=NUM_K)
    return pl.pallas_call(
        kern,
        grid=(1,),
        in_specs=[_const_spec(pos_flat), _const_spec(pos_flat)],
        out_specs=[
            pl.BlockSpec((3, NUM_K, s), lambda t: (0, 0, 0)),
            pl.BlockSpec((NUM_K, s), lambda t: (0, 0)),
        ],
        out_shape=out_shape,
    )(pos_flat, pos_flat)


def _gaff_body(*refs, pre, post, two, stats):
    i = 0
    x_ref = refs[i]; i += 1
    if pre is not None:
        sc_ref, sh_ref = refs[i], refs[i + 1]; i += 2
        z = _act(sc_ref[...] * x_ref[...] + sh_ref[...], pre)
    else:
        z = x_ref[...]
    a_ref = refs[i]; i += 1
    y = _mm(a_ref[...], z)
    if two:
        x2_ref, a2_ref = refs[i], refs[i + 1]; i += 2
        y = y + _mm(a2_ref[...], x2_ref[...])
    c_ref = refs[i]; i += 1
    y = _act(y + c_ref[...], post)
    y_ref = refs[i]; i += 1
    y_ref[...] = y
    if stats:
        s_ref, q_ref = refs[i], refs[i + 1]

        @pl.when(pl.program_id(0) == 0)
        def _():
            s_ref[...] = jnp.zeros_like(s_ref)
            q_ref[...] = jnp.zeros_like(q_ref)
        s, q = _rowsums(y)
        s_ref[...] += s
        q_ref[...] += q


def _gaff_call(x, a, c, pre=None, prevec=None, post=None, x2=None, a2=None,
               stats=False):
    cch, s = x.shape
    o = a.shape[0]
    t = min(_T, s)
    two = x2 is not None
    kern = functools.partial(_gaff_body, pre=pre, post=post, two=two,
                             stats=stats)
    in_specs = [pl.BlockSpec((cch, t), lambda tt: (0, tt))]
    args = [x]
    if pre is not None:
        in_specs += [_const_spec(prevec[0]), _const_spec(prevec[1])]
        args += [prevec[0], prevec[1]]
    in_specs.append(_const_spec(a))
    args.append(a)
    if two:
        in_specs += [pl.BlockSpec((x2.shape[0], t), lambda tt: (0, tt)),
                     _const_spec(a2)]
        args += [x2, a2]
    in_specs.append(_const_spec(c))
    args.append(c)
    out_specs = [pl.BlockSpec((o, t), lambda tt: (0, tt))]
    out_shape = [jax.ShapeDtypeStruct((o, s), jnp.float32)]
    if stats:
        out_specs += [pl.BlockSpec((o, 1), lambda tt: (0, 0)),
                      pl.BlockSpec((o, 1), lambda tt: (0, 0))]
        out_shape += [jax.ShapeDtypeStruct((o, 1), jnp.float32),
                      jax.ShapeDtypeStruct((o, 1), jnp.float32)]
    res = pl.pallas_call(
        kern,
        grid=(s // t,),
        in_specs=in_specs,
        out_specs=out_specs,
        out_shape=out_shape,
    )(*args)
    return res if stats else res[0]


def _norm_body(x_ref, sc_ref, sh_ref, y_ref, *, slope):
    y_ref[...] = _act(sc_ref[...] * x_ref[...] + sh_ref[...], slope)


def _norm_call(x, sc, sh, slope):
    cch, s = x.shape
    t = min(_T, s)
    kern = functools.partial(_norm_body, slope=slope)
    return pl.pallas_call(
        kern,
        grid=(s // t,),
        in_specs=[pl.BlockSpec((cch, t), lambda tt: (0, tt)),
                  _const_spec(sc), _const_spec(sh)],
        out_specs=pl.BlockSpec((cch, t), lambda tt: (0, tt)),
        out_shape=jax.ShapeDtypeStruct((cch, s), jnp.float32),
    )(x, sc, sh)


def _prestats_body(pos_ref, nbrs_ref, d2_ref, feat_ref,
                   wl1_ref, bl1_ref, wl2_ref, bl2_ref, wsc_ref, bsc_ref,
                   s1_ref, q1_ref, s2_ref, q2_ref, ssc_ref, qsc_ref):
    @pl.when(pl.program_id(0) == 0)
    def _():
        for r in (s1_ref, q1_ref, s2_ref, q2_ref, ssc_ref, qsc_ref):
            r[...] = jnp.zeros_like(r)
    ext = pos_ref[...]
    wl1, bl1 = wl1_ref[...], bl1_ref[...]
    wl2, bl2 = wl2_ref[...], bl2_ref[...]
    s1 = jnp.zeros_like(s1_ref)
    q1 = jnp.zeros_like(q1_ref)
    s2 = jnp.zeros_like(s2_ref)
    q2 = jnp.zeros_like(q2_ref)
    for kk in range(NUM_K):
        nb = nbrs_ref[:, kk, :]
        gk = jnp.concatenate([ext, nb, ext - nb, d2_ref[kk:kk + 1, :]], axis=0)
        e1 = _mm(wl1, gk) + bl1
        a, b = _rowsums(e1)
        s1 += a
        q1 += b
        e2 = _mm(wl2, gk) + bl2
        a, b = _rowsums(e2)
        s2 += a
        q2 += b
    s1_ref[...] += s1
    q1_ref[...] += q1
    s2_ref[...] += s2
    q2_ref[...] += q2
    ysc = _mm(wsc_ref[...], feat_ref[...]) + bsc_ref[...]
    a, b = _rowsums(ysc)
    ssc_ref[...] += a
    qsc_ref[...] += b


def _prestats_call(pos_flat, nbrs, d2, feats, wl1, bl1, wl2, bl2, wsc, bsc):
    fch, s = feats.shape
    eg = wl1.shape[0]
    d2ch = wsc.shape[0]
    vec = lambda ch: jax.ShapeDtypeStruct((ch, 1), jnp.float32)
    cvec = lambda ch: pl.BlockSpec((ch, 1), lambda tt: (0, 0))
    return pl.pallas_call(
        _prestats_body,
        grid=(s // _T,),
        in_specs=[
            pl.BlockSpec((3, _T), lambda tt: (0, tt)),
            pl.BlockSpec((3, NUM_K, _T), lambda tt: (0, 0, tt)),
            pl.BlockSpec((NUM_K, _T), lambda tt: (0, tt)),
            pl.BlockSpec((fch, _T), lambda tt: (0, tt)),
            _const_spec(wl1), _const_spec(bl1),
            _const_spec(wl2), _const_spec(bl2),
            _const_spec(wsc), _const_spec(bsc),
        ],
        out_specs=[cvec(eg), cvec(eg), cvec(eg), cvec(eg),
                   cvec(d2ch), cvec(d2ch)],
        out_shape=[vec(eg), vec(eg), vec(eg), vec(eg), vec(d2ch), vec(d2ch)],
    )(pos_flat, nbrs, d2, feats, wl1, bl1, wl2, bl2, wsc, bsc)


def _pool_body(pos_ref, nbrs_ref, d2_ref, feat_ref, f1_ref, f2_ref,
               wl_ref, bl_ref, scl_ref, shl_ref, ws_ref, wp_ref, bp_ref,
               y_ref, s_ref, q_ref, xc_ref, *, mode, slope_f):
    ext = pos_ref[...]
    if mode == 'mm':
        xf = _act(_mm(f1_ref[...], feat_ref[...]) + f2_ref[...], slope_f)
    else:
        xf = _act(f1_ref[...] * feat_ref[...] + f2_ref[...], 0.0)
    wl, bl = wl_ref[...], bl_ref[...]
    scl, shl = scl_ref[...], shl_ref[...]
    ws = ws_ref[...]
    d = ws.shape[0]
    t = ext.shape[1]
    m = jnp.full((d, t), -jnp.inf, jnp.float32)
    for kk in range(NUM_K):
        nb = nbrs_ref[:, kk, :]
        gk = jnp.concatenate([ext, nb, ext - nb, d2_ref[kk:kk + 1, :]], axis=0)
        enc = _act(scl * (_mm(wl, gk) + bl) + shl, 0.0)
        xk = jnp.concatenate([enc, xf], axis=0)
        xc_ref[kk] = xk
        m = jnp.maximum(m, _mm(ws, xk))
    ssum = jnp.zeros((d, t), jnp.float32)
    acc = jnp.zeros((d, t), jnp.float32)
    for kk in range(NUM_K):
        xk = xc_ref[kk]
        e = jnp.exp(_mm(ws, xk) - m)
        ssum += e
        acc += e * xk
    pooled = acc / ssum
    y = _mm(wp_ref[...], pooled) + bp_ref[...]
    y_ref[...] = y

    @pl.when(pl.program_id(0) == 0)
    def _():
        s_ref[...] = jnp.zeros_like(s_ref)
        q_ref[...] = jnp.zeros_like(q_ref)
    s, q = _rowsums(y)
    s_ref[...] += s
    q_ref[...] += q


def _pool_call(pos_flat, nbrs, d2, feat, f1, f2, wl, bl, scl, shl, ws,
               wp, bp, mode, slope_f=None):
    fch, s = feat.shape
    d = ws.shape[0]
    op = wp.shape[0]
    kern = functools.partial(_pool_body, mode=mode, slope_f=slope_f)
    return pl.pallas_call(
        kern,
        grid=(s // _T,),
        in_specs=[
            pl.BlockSpec((3, _T), lambda tt: (0, tt)),
            pl.BlockSpec((3, NUM_K, _T), lambda tt: (0, 0, tt)),
            pl.BlockSpec((NUM_K, _T), lambda tt: (0, tt)),
            pl.BlockSpec((fch, _T), lambda tt: (0, tt)),
            _const_spec(f1), _const_spec(f2),
            _const_spec(wl), _const_spec(bl),
            _const_spec(scl), _const_spec(shl),
            _const_spec(ws), _const_spec(wp), _const_spec(bp),
        ],
        out_specs=[
            pl.BlockSpec((op, _T), lambda tt: (0, tt)),
            pl.BlockSpec((op, 1), lambda tt: (0, 0)),
            pl.BlockSpec((op, 1), lambda tt: (0, 0)),
        ],
        out_shape=[
            jax.ShapeDtypeStruct((op, s), jnp.float32),
            jax.ShapeDtypeStruct((op, 1), jnp.float32),
            jax.ShapeDtypeStruct((op, 1), jnp.float32),
        ],
        scratch_shapes=[pltpu.VMEM((NUM_K, d, _T), jnp.float32)],
    )(pos_flat, nbrs, d2, feat, f1, f2, wl, bl, scl, shl, ws, wp, bp)


def _dual_body(y2_ref, scp_ref, shp_ref, wm_ref, bm_ref, fin_ref,
               wsc_ref, bsc_ref, scs_ref, shs_ref, h_ref):
    p2 = _act(scp_ref[...] * y2_ref[...] + shp_ref[...], 0.0)
    y = _mm(wm_ref[...], p2) + bm_ref[...]
    ysc = scs_ref[...] * (_mm(wsc_ref[...], fin_ref[...]) + bsc_ref[...]) \
        + shs_ref[...]
    h_ref[...] = _act(y + ysc, 0.01)


def _dual_call(y2, scp, shp, wm, bm, fin, wsc, bsc, scs, shs):
    dch, s = y2.shape
    fch = fin.shape[0]
    o = wm.shape[0]
    return pl.pallas_call(
        _dual_body,
        grid=(s // _T,),
        in_specs=[
            pl.BlockSpec((dch, _T), lambda tt: (0, tt)),
            _const_spec(scp), _const_spec(shp),
            _const_spec(wm), _const_spec(bm),
            pl.BlockSpec((fch, _T), lambda tt: (0, tt)),
            _const_spec(wsc), _const_spec(bsc),
            _const_spec(scs), _const_spec(shs),
        ],
        out_specs=pl.BlockSpec((o, _T), lambda tt: (0, tt)),
        out_shape=jax.ShapeDtypeStruct((o, s), jnp.float32),
    )(y2, scp, shp, wm, bm, fin, wsc, bsc, scs, shs)


def _up_body(keys_ref, q_ref, h_ref, *rest, nc, tq, nb, norm):
    if norm:
        sc_ref, sh_ref, up_ref = rest
        h_all = _act(sc_ref[...] * h_ref[...] + sh_ref[...], 0.0)
    else:
        (up_ref,) = rest
        h_all = h_ref[...]
    for b in range(nb):
        kT = keys_ref[:, b * nc:(b + 1) * nc]
        qT = q_ref[:, b * tq:(b + 1) * tq]
        h_b = h_all[:, b * nc:(b + 1) * nc]
        sq_k = jnp.sum(kT * kT, axis=0, keepdims=True)
        sq_q = jnp.sum(qT * qT, axis=0)[:, None]
        cross = jax.lax.dot_general(
            qT, kT, (((0,), (0,)), ((), ())),
            preferred_element_type=jnp.float32)
        d2 = sq_q + sq_k - 2.0 * cross
        m = jnp.min(d2, axis=1, keepdims=True)
        iota = jax.lax.broadcasted_iota(jnp.int32, d2.shape, 1)
        ij = jnp.min(jnp.where(d2 == m, iota, nc), axis=1)
        sel = iota == ij[:, None]
        up = _exact_gather(h_b, sel)
        up_ref[:, b * tq:(b + 1) * tq] = up


def _up_call(pos_s_flat, pos_q_flat, h, b, nc, nf, scsh=None):
    c = h.shape[0]
    sf = b * nf
    norm = scsh is not None
    extra = list(scsh) if norm else []
    out_shape = jax.ShapeDtypeStruct((c, sf), jnp.float32)
    if nf >= _T and nc % 128 == 0:
        nt = nf // _T
        kern = functools.partial(_up_body, nc=nc, tq=_T, nb=1, norm=norm)
        return pl.pallas_call(
            kern,
            grid=(b, nt),
            in_specs=[
                pl.BlockSpec((3, nc), lambda bb, tt: (0, bb)),
                pl.BlockSpec((3, _T), lambda bb, tt: (0, bb * nt + tt)),
                pl.BlockSpec((c, nc), lambda bb, tt: (0, bb)),
            ] + [_const_spec(e) for e in extra],
            out_specs=pl.BlockSpec((c, _T), lambda bb, tt: (0, bb * nt + tt)),
            out_shape=out_shape,
        )(pos_s_flat, pos_q_flat, h, *extra)
    kern = functools.partial(_up_body, nc=nc, tq=nf, nb=b, norm=norm)
    return pl.pallas_call(
        kern,
        grid=(1, 1),
        in_specs=[_const_spec(pos_s_flat), _const_spec(pos_q_flat),
                  _const_spec(h)] + [_const_spec(e) for e in extra],
        out_specs=pl.BlockSpec((c, sf), lambda bb, tt: (0, 0)),
        out_shape=out_shape,
    )(pos_s_flat, pos_q_flat, h, *extra)


def _b2(p):
    return p['b'][:, None]


def _lfa_fused(p, pos_flat, feats, b, n):
    s = b * n
    nbrs, d2 = _knng_call(pos_flat, b, n)
    s1, q1, s2, q2, ssc, qsc = _prestats_call(
        pos_flat, nbrs, d2, feats,
        p['lse1']['W'], _b2(p['lse1']), p['lse2']['W'], _b2(p['lse2']),
        p['shortcut']['W'], _b2(p['shortcut']))
    scl1, shl1 = _bn_vec(p['lse1'], s1, q1, s * NUM_K)
    scl2, shl2 = _bn_vec(p['lse2'], s2, q2, s * NUM_K)
    scsc, shsc = _bn_vec(p['shortcut'], ssc, qsc, s)
    y1, sp1, qp1 = _pool_call(
        pos_flat, nbrs, d2, feats, p['mlp1']['W'], _b2(p['mlp1']),
        p['lse1']['W'], _b2(p['lse1']), scl1, shl1,
        p['pool1_score'], p['pool1_mlp']['W'], _b2(p['pool1_mlp']),
        mode='mm', slope_f=0.2)
    scp1, shp1 = _bn_vec(p['pool1_mlp'], sp1, qp1, s)
    y2, sp2, qp2 = _pool_call(
        pos_flat, nbrs, d2, y1, scp1, shp1,
        p['lse2']['W'], _b2(p['lse2']), scl2, shl2,
        p['pool2_score'], p['pool2_mlp']['W'], _b2(p['pool2_mlp']),
        mode='norm')
    scp2, shp2 = _bn_vec(p['pool2_mlp'], sp2, qp2, s)
    return _dual_call(y2, scp2, shp2, p['mlp2']['W'], _b2(p['mlp2']),
                      feats, p['shortcut']['W'], _b2(p['shortcut']),
                      scsc, shsc)


def kernel(x, params):
    B, N, _ = x.shape
    S = B * N
    coords = jnp.transpose(x[..., :3], (0, 2, 1))
    xT = jnp.transpose(x, (2, 0, 1)).reshape(D_IN, S)

    h_raw, s0, q0 = _gaff_call(
        xT, params['fc_start']['W'], _b2(params['fc_start']), stats=True)
    sc0, sh0 = _bn_vec(params['bn_start'], s0, q0, S)
    h = _norm_call(h_raw, sc0, sh0, 0.2)

    ratio = 1
    stack = []
    pos_flats = []
    for p in params['enc']:
        n_cur = N // ratio
        pos_flat = jnp.transpose(
            coords[:, :, :n_cur], (1, 0, 2)).reshape(3, B * n_cur)
        pos_flats.append(pos_flat)
        h = _lfa_fused(p, pos_flat, h, B, n_cur)
        stack.append(h)
        ratio *= DECIM
        n_new = N // ratio
        ch = h.shape[0]
        h = h.reshape(ch, B, n_cur)[:, :, :n_new].reshape(ch, B * n_new)

    h = _gaff_call(h, params['mid']['W'], _b2(params['mid']), post=0.0)

    scsh = None
    prev_p = None
    for di, p in enumerate(params['dec']):
        n_coarse = N // ratio
        ratio //= DECIM
        n_fine = N // ratio
        pos_s = jnp.transpose(
            coords[:, :, :n_coarse], (1, 0, 2)).reshape(3, B * n_coarse)
        pos_q = pos_flats[3 - di]
        skip = stack.pop()
        up = _up_call(pos_s, pos_q, h, B, n_coarse, n_fine, scsh=scsh)
        c1 = up.shape[0]
        h, sd, qd = _gaff_call(
            up, p['W'][:, :c1], _b2(p), x2=skip, a2=p['W'][:, c1:],
            stats=True)
        scd, shd = _bn_vec(p, sd, qd, B * n_fine)
        scsh = (scd, shd)
        prev_p = p

    h, se, qe = _gaff_call(
        h, params['fc_end1']['W'], _b2(params['fc_end1']),
        pre=0.0, prevec=scsh, stats=True)
    sce, she = _bn_vec(params['fc_end1'], se, qe, S)
    h, se2, qe2 = _gaff_call(
        h, params['fc_end2']['W'], _b2(params['fc_end2']),
        pre=0.0, prevec=(sce, she), stats=True)
    sce2, she2 = _bn_vec(params['fc_end2'], se2, qe2, S)
    out = _gaff_call(
        h, params['fc_out']['W'], _b2(params['fc_out']),
        pre=0.0, prevec=(sce2, she2))
    return jnp.transpose(out.reshape(NUM_CLASSES, B, N), (1, 0, 2))
